# baseline (speedup 1.0000x reference)
import functools

import jax
import jax.numpy as jnp
from jax import lax
from jax.experimental import pallas as pl
from jax.experimental.pallas import tpu as pltpu
from jax.experimental.pallas import tpu_sc as plsc

_TOTAL_WORDS = 100000 * 128
_N_WORKERS = 32
_PER_WORKER = _TOTAL_WORDS // _N_WORKERS
_N_CHUNKS = 8
_CHUNK = _PER_WORKER // _N_CHUNKS


def _sc_copy_body(x_hbm, o_hbm, buf0, buf1, si0, si1, so0, so1):
    wid = lax.axis_index("s") * 2 + lax.axis_index("c")
    base = wid * _PER_WORKER
    bufs = (buf0, buf1)
    si = (si0, si1)
    so = (so0, so1)

    def in_copy(i, b):
        return pltpu.make_async_copy(
            x_hbm.at[pl.ds(base + i * _CHUNK, _CHUNK)], bufs[b], si[b])

    def out_copy(i, b):
        return pltpu.make_async_copy(
            bufs[b], o_hbm.at[pl.ds(base + i * _CHUNK, _CHUNK)], so[b])

    in_copy(0, 0).start()
    for i in range(_N_CHUNKS):
        b = i % 2
        in_copy(i, b).wait()
        if i + 1 < _N_CHUNKS:
            if i >= 1:
                out_copy(i - 1, (i - 1) % 2).wait()
            in_copy(i + 1, (i + 1) % 2).start()
        out_copy(i, b).start()
    out_copy(_N_CHUNKS - 2, (_N_CHUNKS - 2) % 2).wait()
    out_copy(_N_CHUNKS - 1, (_N_CHUNKS - 1) % 2).wait()


@functools.partial(
    pl.kernel,
    out_type=jax.ShapeDtypeStruct((_TOTAL_WORDS,), jnp.float32),
    mesh=plsc.VectorSubcoreMesh(core_axis_name="c", subcore_axis_name="s"),
    scratch_types=[
        pltpu.VMEM((_CHUNK,), jnp.float32),
        pltpu.VMEM((_CHUNK,), jnp.float32),
        pltpu.SemaphoreType.DMA,
        pltpu.SemaphoreType.DMA,
        pltpu.SemaphoreType.DMA,
        pltpu.SemaphoreType.DMA,
    ],
)
def _sc_copy(x_hbm, o_hbm, buf0, buf1, si0, si1, so0, so1):
    _sc_copy_body(x_hbm, o_hbm, buf0, buf1, si0, si1, so0, so1)


def kernel(x):
    return _sc_copy(x.reshape(-1)).reshape(x.shape)


# speedup vs baseline: 1.7876x; 1.7876x over previous
import jax
from jax.experimental import pallas as pl
from jax.experimental.pallas import tpu as pltpu


_BLOCK_ROWS = 19200


def _identity_copy_kernel(x_ref, o_ref):
    o_ref[...] = x_ref[...]


def kernel(x):
    rows = x.shape[0]
    return pl.pallas_call(
        _identity_copy_kernel,
        grid=(pl.cdiv(rows, _BLOCK_ROWS),),
        in_specs=[pl.BlockSpec((_BLOCK_ROWS, x.shape[1]), lambda i: (i, 0))],
        out_specs=pl.BlockSpec((_BLOCK_ROWS, x.shape[1]), lambda i: (i, 0)),
        out_shape=jax.ShapeDtypeStruct(x.shape, x.dtype),
        compiler_params=pltpu.CompilerParams(
            dimension_semantics=("arbitrary",),
        ),
    )(x)
